# baseline (speedup 1.0000x reference)
import functools

import jax
import jax.numpy as jnp
from jax import lax
from jax.experimental import pallas as pl
from jax.experimental.pallas import tpu as pltpu
from jax.experimental.pallas import tpu_sc as plsc

_COLS = 1024
_ROWS_OUT = 65535
_NW = 32
_RPW = 2048
_C = 16
_NCH = _RPW // _C
_LAST_FULL = _ROWS_OUT - 23
_TAIL_START = _ROWS_OUT - 7
_NK = _COLS // 16

_mesh = plsc.VectorSubcoreMesh(core_axis_name="c", subcore_axis_name="s")


@functools.partial(
    pl.kernel,
    mesh=_mesh,
    compiler_params=pltpu.CompilerParams(needs_layout_passes=False),
    out_type=jax.ShapeDtypeStruct((_ROWS_OUT, _COLS + 1), jnp.float32),
    scratch_types=[
        pltpu.VMEM((_C, _COLS), jnp.float32),
        pltpu.VMEM((_C, _COLS), jnp.float32),
        pltpu.VMEM((_C, _COLS + 1), jnp.float32),
        pltpu.VMEM((_C, _COLS + 1), jnp.float32),
        pltpu.SemaphoreType.DMA,
        pltpu.SemaphoreType.DMA,
        pltpu.SemaphoreType.DMA,
        pltpu.SemaphoreType.DMA,
    ],
)
def _sc_scan(x_hbm, out_hbm, in0, in1, o0, o1, si0, si1, so0, so1):
    wid = lax.axis_index("s") * 2 + lax.axis_index("c")
    base = wid * _RPW
    ins = (in0, in1)
    outs = (o0, o1)
    isems = (si0, si1)
    osems = (so0, so1)

    def r0_of(i):
        return pl.multiple_of(jnp.minimum(base + i * _C, _LAST_FULL), 8)

    def scan_rows(ibuf, obuf, nrows):
        lane = lax.iota(jnp.int32, 16)
        last_lane = lane == 15
        col_last = jnp.full((16,), _COLS, jnp.int32)

        def row_body(r, _):
            carry = jnp.float32(0.0)
            for k in range(_NK):
                v = ibuf[r, pl.ds(k * 16, 16)]
                s = plsc.cumsum(v)
                t = jnp.sum(v)
                obuf[r, pl.ds(k * 16, 16)] = (s - v) + carry
                if k == _NK - 1:
                    plsc.store_scatter(
                        obuf, [jnp.full((16,), r, jnp.int32), col_last],
                        s + carry, mask=last_lane)
                carry = carry + t
            return 0

        lax.fori_loop(0, nrows, row_body, 0)

    def chunk(i, _):
        r0 = r0_of(i)
        pltpu.sync_copy(x_hbm.at[pl.ds(r0, _C)], in0)
        scan_rows(in0, o0, _C)
        pltpu.sync_copy(o0.at[pl.ds(0, _C)], out_hbm.at[pl.ds(r0, _C)])
        return 0

    lax.fori_loop(0, _NCH, chunk, 0)

    @pl.when(wid == _NW - 1)
    def _():
        pltpu.sync_copy(x_hbm.at[pl.ds(_TAIL_START, 7)], in0.at[pl.ds(0, 7)])
        scan_rows(in0, o0, 7)
        pltpu.sync_copy(o0.at[pl.ds(0, 7)], out_hbm.at[pl.ds(_TAIL_START, 7)])


def kernel(x):
    return _sc_scan(x)


# speedup vs baseline: 1.1211x; 1.1211x over previous
import functools

import jax
import jax.numpy as jnp
from jax import lax
from jax.experimental import pallas as pl
from jax.experimental.pallas import tpu as pltpu
from jax.experimental.pallas import tpu_sc as plsc

_COLS = 1024
_ROWS_OUT = 65535
_NW = 32
_RPW = 2048
_C = 16
_NCH = _RPW // _C
_LAST_FULL = _ROWS_OUT - 23
_TAIL_START = _ROWS_OUT - 7
_NK = _COLS // 16

_mesh = plsc.VectorSubcoreMesh(core_axis_name="c", subcore_axis_name="s")


@functools.partial(
    pl.kernel,
    mesh=_mesh,
    compiler_params=pltpu.CompilerParams(needs_layout_passes=False),
    out_type=jax.ShapeDtypeStruct((_ROWS_OUT, _COLS + 1), jnp.float32),
    scratch_types=[
        pltpu.VMEM((_C, _COLS), jnp.float32),
        pltpu.VMEM((_C, _COLS), jnp.float32),
        pltpu.VMEM((_C, _COLS + 1), jnp.float32),
        pltpu.VMEM((_C, _COLS + 1), jnp.float32),
        pltpu.SemaphoreType.DMA,
        pltpu.SemaphoreType.DMA,
        pltpu.SemaphoreType.DMA,
        pltpu.SemaphoreType.DMA,
    ],
)
def _sc_scan(x_hbm, out_hbm, in0, in1, o0, o1, si0, si1, so0, so1):
    wid = lax.axis_index("s") * 2 + lax.axis_index("c")
    base = wid * _RPW
    ins = (in0, in1)
    outs = (o0, o1)
    isems = (si0, si1)
    osems = (so0, so1)

    lane = lax.iota(jnp.int32, 16)
    last_lane = lane == 15
    col_last = jnp.full((16,), _COLS, jnp.int32)

    def r0_of(i):
        return pl.multiple_of(jnp.minimum(base + i * _C, _LAST_FULL), 8)

    def do_chunk(ibuf, obuf, r):
        carry = jnp.zeros((16,), jnp.float32)
        for k in range(_NK):
            v = ibuf[r, pl.ds(k * 16, 16)]
            s = plsc.cumsum(v)
            sfx = lax.rev(plsc.cumsum(lax.rev(v, (0,))), (0,))
            e = s - v
            obuf[r, pl.ds(k * 16, 16)] = e + carry
            if k == _NK - 1:
                plsc.store_scatter(
                    obuf, [jnp.full((16,), r, jnp.int32), col_last],
                    s + carry, mask=last_lane)
            carry = carry + (e + sfx)
        return carry

    def scan_rows2(ibuf, obuf):
        def pair_body(q, _):
            do_chunk(ibuf, obuf, 2 * q)
            do_chunk(ibuf, obuf, 2 * q + 1)
            return 0

        lax.fori_loop(0, _C // 2, pair_body, 0)

    def scan_rows1(ibuf, obuf, nrows):
        def row_body(r, _):
            do_chunk(ibuf, obuf, r)
            return 0

        lax.fori_loop(0, nrows, row_body, 0)

    pltpu.async_copy(x_hbm.at[pl.ds(r0_of(0), _C)], ins[0], isems[0])

    def step(i, b):
        pltpu.make_async_copy(
            x_hbm.at[pl.ds(r0_of(i), _C)], ins[b], isems[b]).wait()

        @pl.when(i + 1 < _NCH)
        def _():
            pltpu.async_copy(
                x_hbm.at[pl.ds(r0_of(i + 1), _C)], ins[1 - b], isems[1 - b])

        @pl.when(i >= 2)
        def _():
            pltpu.make_async_copy(
                outs[b], out_hbm.at[pl.ds(r0_of(i - 2), _C)], osems[b]).wait()

        scan_rows2(ins[b], outs[b])
        pltpu.async_copy(outs[b], out_hbm.at[pl.ds(r0_of(i), _C)], osems[b])

    def duo(i0, _):
        for b in range(2):
            step(i0 * 2 + b, b)
        return 0

    lax.fori_loop(0, _NCH // 2, duo, 0)

    pltpu.make_async_copy(
        outs[0], out_hbm.at[pl.ds(r0_of(_NCH - 2), _C)], osems[0]).wait()
    pltpu.make_async_copy(
        outs[1], out_hbm.at[pl.ds(r0_of(_NCH - 1), _C)], osems[1]).wait()

    @pl.when(wid == _NW - 1)
    def _():
        pltpu.sync_copy(x_hbm.at[pl.ds(_TAIL_START, 7)], in0.at[pl.ds(0, 7)])
        scan_rows1(in0, o0, 7)
        pltpu.sync_copy(o0.at[pl.ds(0, 7)], out_hbm.at[pl.ds(_TAIL_START, 7)])


def kernel(x):
    return _sc_scan(x)


# speedup vs baseline: 4.7944x; 4.2764x over previous
import jax
import jax.numpy as jnp
from jax.experimental import pallas as pl

_COLS = 1024
_ROWS_OUT = 65535
_BLK_R = 2048


def _scan_kernel(x_ref, u_ref, o_ref):
    x = x_ref[...]
    xb = x.astype(jnp.bfloat16)
    excl = jax.lax.dot_general(
        xb, u_ref[...],
        dimension_numbers=(((1,), (0,)), ((), ())),
        preferred_element_type=jnp.float32,
    )
    o_ref[:, :_COLS] = excl
    o_ref[:, _COLS:] = excl[:, _COLS - 1:_COLS] + x[:, _COLS - 1:_COLS]


def kernel(x):
    col = jax.lax.broadcasted_iota(jnp.int32, (_COLS, _COLS), 1)
    row = jax.lax.broadcasted_iota(jnp.int32, (_COLS, _COLS), 0)
    u_strict = (row < col).astype(jnp.bfloat16)
    grid = (pl.cdiv(_ROWS_OUT, _BLK_R),)
    return pl.pallas_call(
        _scan_kernel,
        grid=grid,
        in_specs=[
            pl.BlockSpec((_BLK_R, _COLS), lambda i: (i, 0)),
            pl.BlockSpec((_COLS, _COLS), lambda i: (0, 0)),
        ],
        out_specs=pl.BlockSpec((_BLK_R, _COLS + 1), lambda i: (i, 0)),
        out_shape=jax.ShapeDtypeStruct((_ROWS_OUT, _COLS + 1), x.dtype),
    )(x, u_strict)


# speedup vs baseline: 4.8264x; 1.0067x over previous
import jax
import jax.numpy as jnp
from jax.experimental import pallas as pl
from jax.experimental.pallas import tpu as pltpu

_COLS = 1024
_ROWS_OUT = 65535
_BLK_R = 3072


def _scan_kernel(x_ref, u_ref, o_ref):
    x = x_ref[...]
    xb = x.astype(jnp.bfloat16)
    excl = jax.lax.dot_general(
        xb, u_ref[...],
        dimension_numbers=(((1,), (0,)), ((), ())),
        preferred_element_type=jnp.float32,
    )
    o_ref[:, :_COLS] = excl
    o_ref[:, _COLS:] = excl[:, _COLS - 1:_COLS] + x[:, _COLS - 1:_COLS]


def kernel(x):
    col = jax.lax.broadcasted_iota(jnp.int32, (_COLS, _COLS), 1)
    row = jax.lax.broadcasted_iota(jnp.int32, (_COLS, _COLS), 0)
    u_strict = (row < col).astype(jnp.bfloat16)
    grid = (pl.cdiv(_ROWS_OUT, _BLK_R),)
    return pl.pallas_call(
        _scan_kernel,
        grid=grid,
        in_specs=[
            pl.BlockSpec((_BLK_R, _COLS), lambda i: (i, 0)),
            pl.BlockSpec((_COLS, _COLS), lambda i: (0, 0)),
        ],
        out_specs=pl.BlockSpec((_BLK_R, _COLS + 1), lambda i: (i, 0)),
        out_shape=jax.ShapeDtypeStruct((_ROWS_OUT, _COLS + 1), x.dtype),
        compiler_params=pltpu.CompilerParams(vmem_limit_bytes=100 * 1024 * 1024),
    )(x, u_strict)
